# baseline (speedup 1.0000x reference)
import functools

import jax
import jax.numpy as jnp
from jax import lax
from jax.experimental import pallas as pl
from jax.experimental.pallas import tpu as pltpu
from jax.experimental.pallas import tpu_sc as plsc

_NC = 2
_NS = 16
_NW = _NC * _NS
_L = 16


@functools.lru_cache(maxsize=None)
def _build_sc_call(B, N, M):
    E = B * M
    NT = B * N
    GPW = 4
    EW = B // GPW
    assert B == EW * GPW and M % _L == 0 and (M * 2) % 8 == 0
    NPC = 320
    NREM = NT - NPC * (_NW - 1)
    assert 0 < NREM <= NPC and NPC % _L == 0 and NREM % 8 == 0
    NVEC = NPC // _L
    LENB = ((B + _L - 1) // _L) * _L

    mesh = plsc.VectorSubcoreMesh(core_axis_name="c", subcore_axis_name="s")

    @functools.partial(
        pl.kernel,
        mesh=mesh,
        compiler_params=pltpu.CompilerParams(
            needs_layout_passes=False, use_tc_tiling_on_sc=False),
        out_type=[
            jax.ShapeDtypeStruct((2 * E,), jnp.int32),
            jax.ShapeDtypeStruct((E,), jnp.int32),
            jax.ShapeDtypeStruct((E,), jnp.int32),
            jax.ShapeDtypeStruct((NT,), jnp.int32),
            jax.ShapeDtypeStruct((NT,), jnp.int32),
            jax.ShapeDtypeStruct((B,), jnp.int32),
            jax.ShapeDtypeStruct((B,), jnp.int32),
        ],
        scratch_types=(
            [pltpu.VMEM((M, 2), jnp.int32) for _ in range(2)]
            + [pltpu.VMEM((M,), jnp.int32) for _ in range(2)]
            + [pltpu.VMEM((M,), jnp.int32) for _ in range(2)]
            + [pltpu.VMEM((M,), jnp.int32) for _ in range(2)]
            + [
                pltpu.VMEM((M,), jnp.int32),
                pltpu.VMEM((NPC,), jnp.int32),
                pltpu.VMEM((NPC,), jnp.int32),
                pltpu.SemaphoreType.DMA,
                pltpu.SemaphoreType.DMA,
                pltpu.SemaphoreType.DMA,
            ]
        ),
    )
    def sc_fn(ei_hbm, dj_hbm, gie_hbm, eid_hbm, gin_hbm, nid_hbm,
              nl_hbm, el_hbm, *refs):
        inb = refs[0:2]
        dj0b = refs[2:4]
        dj1b = refs[4:6]
        gieb = refs[6:8]
        ebuf, gnb, nnb, sem_in, sem_small, sem_out = refs[8:]

        wid = lax.axis_index("s") * _NC + lax.axis_index("c")
        iota = lax.iota(jnp.int32, _L)
        col0 = iota * 0
        col1 = col0 + 1

        @pl.when(wid < EW)
        def _edge_work():
            b0 = wid * GPW
            cins = [
                pltpu.async_copy(ei_hbm.at[b0 + k], inb[k], sem_in)
                for k in range(2)
            ]

            def eid_body(j, _):
                ebuf[pl.ds(j * _L, _L)] = j * _L + iota
                return 0

            lax.fori_loop(0, M // _L, eid_body, 0)

            couts = []
            for k in range(GPW):
                s = k % 2
                cins[k].wait()
                b = b0 + k
                off = b * N
                if k >= 2:
                    for c in couts[4 * (k - 2):4 * (k - 1)]:
                        c.wait()

                def edge_body(j, _, s=s, off=off, b=b):
                    rows = j * _L + iota
                    src = plsc.load_gather(inb[s], [rows, col0])
                    dst = plsc.load_gather(inb[s], [rows, col1])
                    dj0b[s][pl.ds(j * _L, _L)] = src + off
                    dj1b[s][pl.ds(j * _L, _L)] = dst + off
                    gieb[s][pl.ds(j * _L, _L)] = jnp.broadcast_to(b, (_L,))
                    return 0

                lax.fori_loop(0, M // _L, edge_body, 0)
                if k + 2 < GPW:
                    cins.append(pltpu.async_copy(ei_hbm.at[b0 + k + 2],
                                                 inb[s], sem_in))
                couts += [
                    pltpu.async_copy(dj0b[s], dj_hbm.at[pl.ds(b * M, M)],
                                     sem_out),
                    pltpu.async_copy(dj1b[s], dj_hbm.at[pl.ds(E + b * M, M)],
                                     sem_out),
                    pltpu.async_copy(gieb[s], gie_hbm.at[pl.ds(b * M, M)],
                                     sem_out),
                    pltpu.async_copy(ebuf, eid_hbm.at[pl.ds(b * M, M)],
                                     sem_out),
                ]
            for c in couts[4 * (GPW - 2):]:
                c.wait()

        nbase = wid * NPC

        def node_body(j, _):
            v = (nbase + j * _L) + iota
            gg = v // N
            gnb[pl.ds(j * _L, _L)] = gg
            nnb[pl.ds(j * _L, _L)] = v - gg * N
            return 0

        lax.fori_loop(0, NVEC, node_body, 0)

        @pl.when(wid < _NW - 1)
        def _node_full():
            c0 = pltpu.async_copy(gnb, gin_hbm.at[pl.ds(nbase, NPC)],
                                  sem_small)
            c1 = pltpu.async_copy(nnb, nid_hbm.at[pl.ds(nbase, NPC)],
                                  sem_small)
            c0.wait()
            c1.wait()

        @pl.when(wid == _NW - 1)
        def _node_rem():
            c0 = pltpu.async_copy(gnb.at[pl.ds(0, NREM)],
                                  gin_hbm.at[pl.ds(nbase, NREM)], sem_small)
            c1 = pltpu.async_copy(nnb.at[pl.ds(0, NREM)],
                                  nid_hbm.at[pl.ds(nbase, NREM)], sem_small)
            c0.wait()
            c1.wait()

        @pl.when(wid == EW)
        def _node_len():
            for j in range(LENB // _L):
                ebuf[pl.ds(j * _L, _L)] = jnp.full((_L,), N, jnp.int32)
            pltpu.async_copy(ebuf.at[pl.ds(0, B)], nl_hbm, sem_small).wait()

        @pl.when(wid == EW + 1)
        def _edge_len():
            for j in range(LENB // _L):
                ebuf[pl.ds(j * _L, _L)] = jnp.full((_L,), M, jnp.int32)
            pltpu.async_copy(ebuf.at[pl.ds(0, B)], el_hbm, sem_small).wait()

    return sc_fn


def kernel(nodes, edge_indices):
    B, N, F = nodes.shape
    _, M, _ = edge_indices.shape
    E = B * M

    nodes_flatten = nodes.reshape(B * N, F)
    ei = edge_indices.astype(jnp.int32)

    sc_fn = _build_sc_call(B, N, M)
    dj_flat, gie, eid, gin, nid, nl, el = sc_fn(ei)

    disjoint_indices = dj_flat.reshape(2, E).astype(edge_indices.dtype)
    return (nodes_flatten, disjoint_indices, gin, gie, nid, eid, nl, el)


# speedup vs baseline: 9.2900x; 9.2900x over previous
import functools

import jax
import jax.numpy as jnp
from jax import lax
from jax.experimental import pallas as pl
from jax.experimental.pallas import tpu as pltpu
from jax.experimental.pallas import tpu_sc as plsc

_NC = 2
_NS = 16
_NW = _NC * _NS
_L = 16


@functools.lru_cache(maxsize=None)
def _build_sc_call(B, N, M):
    E = B * M
    NT = B * N
    W = 2 * E
    WPW = W // _NW
    EPW = E // _NW
    GW = 2 * M
    assert W % _NW == 0 and WPW % _L == 0 and WPW % 8 == 0
    assert E % _NW == 0 and EPW % _L == 0 and EPW % 8 == 0
    assert GW % _L == 0 and M % _L == 0
    NPC = 320
    NREM = NT - NPC * (_NW - 1)
    assert 0 < NREM <= NPC and NPC % _L == 0 and NREM % 8 == 0
    NVEC = NPC // _L
    LENB = ((B + _L - 1) // _L) * _L

    mesh = plsc.VectorSubcoreMesh(core_axis_name="c", subcore_axis_name="s")

    @functools.partial(
        pl.kernel,
        mesh=mesh,
        compiler_params=pltpu.CompilerParams(
            needs_layout_passes=False, use_tc_tiling_on_sc=False),
        out_type=[
            jax.ShapeDtypeStruct((W,), jnp.int32),
            jax.ShapeDtypeStruct((E,), jnp.int32),
            jax.ShapeDtypeStruct((E,), jnp.int32),
            jax.ShapeDtypeStruct((NT,), jnp.int32),
            jax.ShapeDtypeStruct((NT,), jnp.int32),
            jax.ShapeDtypeStruct((B,), jnp.int32),
            jax.ShapeDtypeStruct((B,), jnp.int32),
        ],
        scratch_types=[
            pltpu.VMEM((WPW,), jnp.int32),
            pltpu.VMEM((WPW,), jnp.int32),
            pltpu.VMEM((EPW,), jnp.int32),
            pltpu.VMEM((EPW,), jnp.int32),
            pltpu.VMEM((NPC,), jnp.int32),
            pltpu.VMEM((NPC,), jnp.int32),
            pltpu.VMEM((LENB,), jnp.int32),
            pltpu.SemaphoreType.DMA,
            pltpu.SemaphoreType.DMA,
            pltpu.SemaphoreType.DMA,
        ],
    )
    def sc_fn(y_hbm, z_hbm, gie_hbm, eid_hbm, gin_hbm, nid_hbm,
              nl_hbm, el_hbm, inb, zb, gieb, eidb, gnb, nnb, lenb,
              sem_in, sem_small, sem_out):
        wid = lax.axis_index("s") * _NC + lax.axis_index("c")
        iota = lax.iota(jnp.int32, _L)

        wbase = wid * WPW
        cin = pltpu.async_copy(y_hbm.at[pl.ds(wbase, WPW)], inb, sem_in)

        ebase = wid * EPW

        def id_body(j, _):
            e0 = ebase + j * _L
            g = e0 // M
            gieb[pl.ds(j * _L, _L)] = jnp.broadcast_to(g, (_L,))
            eidb[pl.ds(j * _L, _L)] = (e0 - g * M) + iota
            return 0

        lax.fori_loop(0, EPW // _L, id_body, 0)
        cid0 = pltpu.async_copy(gieb, gie_hbm.at[pl.ds(ebase, EPW)], sem_out)
        cid1 = pltpu.async_copy(eidb, eid_hbm.at[pl.ds(ebase, EPW)], sem_out)

        nbase = wid * NPC

        def node_body(j, _):
            v = (nbase + j * _L) + iota
            gg = v // N
            gnb[pl.ds(j * _L, _L)] = gg
            nnb[pl.ds(j * _L, _L)] = v - gg * N
            return 0

        lax.fori_loop(0, NVEC, node_body, 0)

        @pl.when(wid < _NW - 1)
        def _node_full():
            c0 = pltpu.async_copy(gnb, gin_hbm.at[pl.ds(nbase, NPC)],
                                  sem_small)
            c1 = pltpu.async_copy(nnb, nid_hbm.at[pl.ds(nbase, NPC)],
                                  sem_small)
            c0.wait()
            c1.wait()

        @pl.when(wid == _NW - 1)
        def _node_rem():
            c0 = pltpu.async_copy(gnb.at[pl.ds(0, NREM)],
                                  gin_hbm.at[pl.ds(nbase, NREM)], sem_small)
            c1 = pltpu.async_copy(nnb.at[pl.ds(0, NREM)],
                                  nid_hbm.at[pl.ds(nbase, NREM)], sem_small)
            c0.wait()
            c1.wait()

        @pl.when(wid == 0)
        def _node_len():
            for j in range(LENB // _L):
                lenb[pl.ds(j * _L, _L)] = jnp.full((_L,), N, jnp.int32)
            pltpu.async_copy(lenb.at[pl.ds(0, B)], nl_hbm, sem_small).wait()

        @pl.when(wid == 1)
        def _edge_len():
            for j in range(LENB // _L):
                lenb[pl.ds(j * _L, _L)] = jnp.full((_L,), M, jnp.int32)
            pltpu.async_copy(lenb.at[pl.ds(0, B)], el_hbm, sem_small).wait()

        cin.wait()

        def add_body(j, _):
            w0 = wbase + j * _L
            off = (w0 // GW) * N
            zb[pl.ds(j * _L, _L)] = inb[pl.ds(j * _L, _L)] + off
            return 0

        lax.fori_loop(0, WPW // _L, add_body, 0)

        cz = pltpu.async_copy(zb, z_hbm.at[pl.ds(wbase, WPW)], sem_out)
        cid0.wait()
        cid1.wait()
        cz.wait()

    return sc_fn


def kernel(nodes, edge_indices):
    B, N, F = nodes.shape
    _, M, _ = edge_indices.shape
    E = B * M
    idt = edge_indices.dtype

    nodes_flatten = nodes.reshape(B * N, F)

    y = (edge_indices.astype(jnp.int32)
         .reshape(B, M // 128, 128, 2)
         .transpose(0, 1, 3, 2)
         .reshape(-1))

    sc_fn = _build_sc_call(B, N, M)
    z, gie, eid, gin, nid, nl, el = sc_fn(y)

    disjoint_indices = (z.reshape(B * M // 128, 2, 128)
                        .transpose(1, 0, 2)
                        .reshape(2, E)
                        .astype(idt))
    return (nodes_flatten, disjoint_indices, gin, gie, nid, eid, nl, el)


# speedup vs baseline: 9.4673x; 1.0191x over previous
import functools

import jax
import jax.numpy as jnp
from jax import lax
from jax.experimental import pallas as pl
from jax.experimental.pallas import tpu as pltpu
from jax.experimental.pallas import tpu_sc as plsc

_NC = 2
_NS = 16
_NW = _NC * _NS
_L = 16


@functools.lru_cache(maxsize=None)
def _build_sc_call(B, N, M):
    E = B * M
    NT = B * N
    W = 2 * E
    WPW = W // _NW
    EPW = E // _NW
    GW = 2 * M
    assert W % _NW == 0 and WPW % _L == 0 and WPW % 8 == 0
    assert E % _NW == 0 and EPW % _L == 0 and EPW % 8 == 0
    assert GW % _L == 0 and M % _L == 0
    NPC = 320
    NREM = NT - NPC * (_NW - 1)
    assert 0 < NREM <= NPC and NPC % _L == 0 and NREM % 8 == 0
    NVEC = NPC // _L
    LENB = ((B + _L - 1) // _L) * _L

    mesh = plsc.VectorSubcoreMesh(core_axis_name="c", subcore_axis_name="s")

    @functools.partial(
        pl.kernel,
        mesh=mesh,
        compiler_params=pltpu.CompilerParams(
            needs_layout_passes=False, use_tc_tiling_on_sc=False),
        out_type=[
            jax.ShapeDtypeStruct((W,), jnp.int32),
            jax.ShapeDtypeStruct((E,), jnp.int32),
            jax.ShapeDtypeStruct((E,), jnp.int32),
            jax.ShapeDtypeStruct((NT,), jnp.int32),
            jax.ShapeDtypeStruct((NT,), jnp.int32),
            jax.ShapeDtypeStruct((B,), jnp.int32),
            jax.ShapeDtypeStruct((B,), jnp.int32),
        ],
        scratch_types=[
            pltpu.VMEM((WPW,), jnp.int32),
            pltpu.VMEM((WPW,), jnp.int32),
            pltpu.VMEM((EPW,), jnp.int32),
            pltpu.VMEM((EPW,), jnp.int32),
            pltpu.VMEM((NPC,), jnp.int32),
            pltpu.VMEM((NPC,), jnp.int32),
            pltpu.VMEM((LENB,), jnp.int32),
            pltpu.SemaphoreType.DMA,
            pltpu.SemaphoreType.DMA,
            pltpu.SemaphoreType.DMA,
        ],
    )
    def sc_fn(y_hbm, z_hbm, gie_hbm, eid_hbm, gin_hbm, nid_hbm,
              nl_hbm, el_hbm, inb, zb, gieb, eidb, gnb, nnb, lenb,
              sem_in, sem_small, sem_out):
        wid = lax.axis_index("s") * _NC + lax.axis_index("c")
        iota = lax.iota(jnp.int32, _L)

        NSL = 4
        SLW = WPW // NSL
        wbase = wid * WPW
        cins = [
            pltpu.async_copy(
                y_hbm.at[pl.ds(wbase + s * SLW, SLW)],
                inb.at[pl.ds(s * SLW, SLW)], sem_in)
            for s in range(NSL)
        ]

        ebase = wid * EPW

        def id_body(j, _):
            e0 = ebase + j * _L
            g = e0 // M
            gieb[pl.ds(j * _L, _L)] = jnp.broadcast_to(g, (_L,))
            eidb[pl.ds(j * _L, _L)] = (e0 - g * M) + iota
            return 0

        lax.fori_loop(0, EPW // _L, id_body, 0)
        cid0 = pltpu.async_copy(gieb, gie_hbm.at[pl.ds(ebase, EPW)], sem_out)
        cid1 = pltpu.async_copy(eidb, eid_hbm.at[pl.ds(ebase, EPW)], sem_out)

        nbase = wid * NPC

        def node_body(j, _):
            v = (nbase + j * _L) + iota
            gg = v // N
            gnb[pl.ds(j * _L, _L)] = gg
            nnb[pl.ds(j * _L, _L)] = v - gg * N
            return 0

        lax.fori_loop(0, NVEC, node_body, 0)

        @pl.when(wid < _NW - 1)
        def _node_full():
            c0 = pltpu.async_copy(gnb, gin_hbm.at[pl.ds(nbase, NPC)],
                                  sem_small)
            c1 = pltpu.async_copy(nnb, nid_hbm.at[pl.ds(nbase, NPC)],
                                  sem_small)
            c0.wait()
            c1.wait()

        @pl.when(wid == _NW - 1)
        def _node_rem():
            c0 = pltpu.async_copy(gnb.at[pl.ds(0, NREM)],
                                  gin_hbm.at[pl.ds(nbase, NREM)], sem_small)
            c1 = pltpu.async_copy(nnb.at[pl.ds(0, NREM)],
                                  nid_hbm.at[pl.ds(nbase, NREM)], sem_small)
            c0.wait()
            c1.wait()

        @pl.when(wid == 0)
        def _node_len():
            for j in range(LENB // _L):
                lenb[pl.ds(j * _L, _L)] = jnp.full((_L,), N, jnp.int32)
            pltpu.async_copy(lenb.at[pl.ds(0, B)], nl_hbm, sem_small).wait()

        @pl.when(wid == 1)
        def _edge_len():
            for j in range(LENB // _L):
                lenb[pl.ds(j * _L, _L)] = jnp.full((_L,), M, jnp.int32)
            pltpu.async_copy(lenb.at[pl.ds(0, B)], el_hbm, sem_small).wait()

        czs = []
        for s in range(NSL):
            cins[s].wait()

            def add_body(j, _, s=s):
                w0 = wbase + s * SLW + j * _L
                off = (w0 // GW) * N
                zb[pl.ds(s * SLW + j * _L, _L)] = (
                    inb[pl.ds(s * SLW + j * _L, _L)] + off)
                return 0

            lax.fori_loop(0, SLW // _L, add_body, 0)
            czs.append(pltpu.async_copy(
                zb.at[pl.ds(s * SLW, SLW)],
                z_hbm.at[pl.ds(wbase + s * SLW, SLW)], sem_out))

        cid0.wait()
        cid1.wait()
        for c in czs:
            c.wait()

    return sc_fn


def kernel(nodes, edge_indices):
    B, N, F = nodes.shape
    _, M, _ = edge_indices.shape
    E = B * M
    idt = edge_indices.dtype

    nodes_flatten = nodes.reshape(B * N, F)

    y = (edge_indices.astype(jnp.int32)
         .reshape(B, M // 128, 128, 2)
         .transpose(0, 1, 3, 2)
         .reshape(-1))

    sc_fn = _build_sc_call(B, N, M)
    z, gie, eid, gin, nid, nl, el = sc_fn(y)

    disjoint_indices = (z.reshape(B * M // 128, 2, 128)
                        .transpose(1, 0, 2)
                        .reshape(2, E)
                        .astype(idt))
    return (nodes_flatten, disjoint_indices, gin, gie, nid, eid, nl, el)
